# baseline (speedup 1.0000x reference)
import jax
import jax.numpy as jnp
from jax.experimental import pallas as pl
from jax.experimental.pallas import tpu as pltpu

POP = 4096
KEY_DIM = 1024
HIDDEN = 64
CHUNK = 256
NBUF = 4
NCHUNK = POP // CHUNK


def _brkga_fitness_kernel(x_hbm, w_ref, out_ref, *scratch):
    bufs = scratch[:NBUF]
    sems = scratch[NBUF:]

    def copy_in(chunk_idx, slot):
        return pltpu.make_async_copy(
            x_hbm.at[pl.ds(chunk_idx * CHUNK, CHUNK), :],
            bufs[slot],
            sems[slot],
        )

    for b in range(NBUF):
        copy_in(b, b).start()

    w = w_ref[...]
    for i in range(NCHUNK):
        slot = i % NBUF
        copy_in(i, slot).wait()
        h = jnp.dot(bufs[slot][...], w, preferred_element_type=jnp.float32)
        out_ref[pl.ds(i * CHUNK, CHUNK), :] = jnp.sum(
            jnp.maximum(h, 0.0), axis=1, keepdims=True
        )
        nxt = i + NBUF
        if nxt < NCHUNK:
            copy_in(nxt, slot).start()


def kernel(keys_pop, W):
    out = pl.pallas_call(
        _brkga_fitness_kernel,
        in_specs=[
            pl.BlockSpec(memory_space=pl.ANY),
            pl.BlockSpec(memory_space=pltpu.VMEM),
        ],
        out_specs=pl.BlockSpec(memory_space=pltpu.VMEM),
        out_shape=jax.ShapeDtypeStruct((POP, 1), jnp.float32),
        scratch_shapes=[pltpu.VMEM((CHUNK, KEY_DIM), jnp.float32)
                        for _ in range(NBUF)]
        + [pltpu.SemaphoreType.DMA for _ in range(NBUF)],
    )(keys_pop, W)
    return out.reshape(POP)


# speedup vs baseline: 1.0882x; 1.0882x over previous
import jax
import jax.numpy as jnp
from jax.experimental import pallas as pl
from jax.experimental.pallas import tpu as pltpu

POP = 4096
KEY_DIM = 1024
HIDDEN = 64
BLOCK = 512


def _brkga_fitness_kernel(x_ref, w_ref, out_ref):
    h = jnp.dot(x_ref[...], w_ref[...], preferred_element_type=jnp.float32)
    out_ref[...] = jnp.sum(jnp.maximum(h, 0.0), axis=1, keepdims=True)


def kernel(keys_pop, W):
    grid = (POP // BLOCK,)
    out = pl.pallas_call(
        _brkga_fitness_kernel,
        grid=grid,
        in_specs=[
            pl.BlockSpec((BLOCK, KEY_DIM), lambda i: (i, 0)),
            pl.BlockSpec((KEY_DIM, HIDDEN), lambda i: (0, 0)),
        ],
        out_specs=pl.BlockSpec((BLOCK, 1), lambda i: (i, 0)),
        out_shape=jax.ShapeDtypeStruct((POP, 1), jnp.float32),
        compiler_params=pltpu.CompilerParams(
            dimension_semantics=("arbitrary",),
        ),
    )(keys_pop, W)
    return out.reshape(POP)


# speedup vs baseline: 1.2663x; 1.1637x over previous
import jax
import jax.numpy as jnp
from jax.experimental import pallas as pl
from jax.experimental.pallas import tpu as pltpu

POP = 4096
KEY_DIM = 1024
HIDDEN = 64
BLOCK = 1024


def _brkga_fitness_kernel(x_ref, w_ref, out_ref):
    h = jnp.dot(x_ref[...], w_ref[...], preferred_element_type=jnp.float32)
    out_ref[...] = jnp.sum(jnp.maximum(h, 0.0), axis=1, keepdims=True)


def kernel(keys_pop, W):
    grid = (POP // BLOCK,)
    out = pl.pallas_call(
        _brkga_fitness_kernel,
        grid=grid,
        in_specs=[
            pl.BlockSpec((BLOCK, KEY_DIM), lambda i: (i, 0)),
            pl.BlockSpec((KEY_DIM, HIDDEN), lambda i: (0, 0)),
        ],
        out_specs=pl.BlockSpec((BLOCK, 1), lambda i: (i, 0)),
        out_shape=jax.ShapeDtypeStruct((POP, 1), jnp.float32),
        compiler_params=pltpu.CompilerParams(
            dimension_semantics=("arbitrary",),
            skip_device_barrier=True,
            disable_bounds_checks=True,
            disable_semaphore_checks=True,
        ),
    )(keys_pop, W)
    return out.reshape(POP)
